# baseline (speedup 1.0000x reference)
import functools

import jax
import jax.numpy as jnp
from jax import lax
from jax.experimental import pallas as pl
from jax.experimental.pallas import tpu as pltpu
from jax.experimental.pallas import tpu_sc as plsc

_B, _T, _D, _NNEG = 2, 2048, 768, 10
_BT = _B * _T
_NR = _NNEG * _B * _T
_NC, _NS = 2, 16
_NW = _NC * _NS
_C = 80
_TPW = _BT // _NW
_NPW = _NR // _NW
_NCH = _NPW // _C


@functools.partial(
    pl.kernel,
    out_type=(
        jax.ShapeDtypeStruct((_BT, _D), jnp.float32),
        jax.ShapeDtypeStruct((_NR, _D), jnp.float32),
    ),
    mesh=plsc.VectorSubcoreMesh(core_axis_name="c", subcore_axis_name="s"),
    scratch_types=(
        pltpu.VMEM((_TPW,), jnp.int32),
        pltpu.VMEM((_NPW,), jnp.int32),
        pltpu.VMEM((_C, _D), jnp.float32),
        pltpu.VMEM((_C, _D), jnp.float32),
        pltpu.SemaphoreType.DMA,
        pltpu.SemaphoreType.DMA,
    ),
)
def _sc_gather(x_hbm, idxt_hbm, idxn_hbm, tgt_hbm, neg_hbm,
               idxt_v, idxn_v, buf0, buf1, sem0, sem1):
    wid = lax.axis_index("s") * _NC + lax.axis_index("c")
    tbase = wid * _TPW
    nbase = wid * _NPW

    pltpu.sync_copy(idxt_hbm.at[pl.ds(tbase, _TPW)], idxt_v)
    pltpu.sync_copy(idxn_hbm.at[pl.ds(nbase, _NPW)], idxn_v)

    def ngather(c, buf, sem):
        pltpu.async_copy(x_hbm.at[idxn_v.at[pl.ds(c * _C, _C)]], buf, sem)

    def nwait(c, buf, sem):
        pltpu.make_async_copy(x_hbm.at[idxn_v.at[pl.ds(c * _C, _C)]], buf, sem).wait()

    _TC2 = _TPW // 2
    pltpu.async_copy(x_hbm.at[idxt_v.at[pl.ds(0, _TC2)]],
                     buf0.at[pl.ds(0, _TC2)], sem0)
    pltpu.async_copy(x_hbm.at[idxt_v.at[pl.ds(_TC2, _TC2)]],
                     buf1.at[pl.ds(0, _TC2)], sem1)
    pltpu.make_async_copy(x_hbm.at[idxt_v.at[pl.ds(0, _TC2)]],
                          buf0.at[pl.ds(0, _TC2)], sem0).wait()
    pltpu.sync_copy(buf0.at[pl.ds(0, _TC2)], tgt_hbm.at[pl.ds(tbase, _TC2)])
    ngather(0, buf0, sem0)
    pltpu.make_async_copy(x_hbm.at[idxt_v.at[pl.ds(_TC2, _TC2)]],
                          buf1.at[pl.ds(0, _TC2)], sem1).wait()
    pltpu.sync_copy(buf1.at[pl.ds(0, _TC2)],
                    tgt_hbm.at[pl.ds(tbase + _TC2, _TC2)])

    def nbody(k, carry):
        c0 = 2 * k
        c1 = c0 + 1
        ngather(c1, buf1, sem1)
        nwait(c0, buf0, sem0)
        pltpu.sync_copy(buf0, neg_hbm.at[pl.ds(nbase + c0 * _C, _C)])

        @pl.when(c1 + 1 < _NCH)
        def _():
            ngather(c1 + 1, buf0, sem0)

        nwait(c1, buf1, sem1)
        pltpu.sync_copy(buf1, neg_hbm.at[pl.ds(nbase + c1 * _C, _C)])
        return carry

    lax.fori_loop(0, _NCH // 2, nbody, 0)


def _tc_copy_body(x_hbm, xcopy_hbm, sem):
    pltpu.make_async_copy(x_hbm, xcopy_hbm, sem).start()
    pltpu.make_async_copy(x_hbm, xcopy_hbm, sem).wait()


_tc_copy = pl.pallas_call(
    _tc_copy_body,
    out_shape=jax.ShapeDtypeStruct((_BT, _D), jnp.float32),
    in_specs=[pl.BlockSpec(memory_space=pl.ANY)],
    out_specs=pl.BlockSpec(memory_space=pl.ANY),
    scratch_shapes=[pltpu.SemaphoreType.DMA],
)


def kernel(x):
    B, T, D = x.shape
    tszs = jnp.repeat(jnp.arange(T), _NNEG)
    neg = jax.random.randint(jax.random.key(42), (B, _NNEG * T), 0, T - 1)
    neg = jnp.where(neg >= tszs[None, :], neg + 1, neg)
    src_t = jnp.where(neg == T - 1, 0, neg + 1)
    src = src_t + jnp.arange(B)[:, None] * T
    idxn = src.reshape(B, T, _NNEG).transpose(2, 0, 1).reshape(-1)
    idxn = idxn.astype(jnp.int32)
    j = jnp.arange(_BT)
    idxt = jnp.where(j % T == T - 1, j - (T - 1), j + 1).astype(jnp.int32)

    x_flat = x.reshape(_BT, D)
    tgt, negs = _sc_gather(x_flat, idxt, idxn)
    xc = _tc_copy(x_flat)
    return (xc.reshape(B, T, D), tgt.reshape(B, T, D),
            negs.reshape(_NNEG, B, T, D))


# speedup vs baseline: 3.1399x; 3.1399x over previous
import functools

import jax
import jax.numpy as jnp
from jax import lax
from jax.experimental import pallas as pl
from jax.experimental.pallas import tpu as pltpu
from jax.experimental.pallas import tpu_sc as plsc

_B, _T, _D, _NNEG = 2, 2048, 768, 10
_BT = _B * _T
_NR = _NNEG * _B * _T
_NC, _NS = 2, 16
_NW = _NC * _NS
_C = 80
_NPW = _NR // _NW
_NCH = _NPW // _C


@functools.partial(
    pl.kernel,
    out_type=jax.ShapeDtypeStruct((_NR, _D), jnp.float32),
    mesh=plsc.VectorSubcoreMesh(core_axis_name="c", subcore_axis_name="s"),
    scratch_types=(
        pltpu.VMEM((_NPW,), jnp.int32),
        pltpu.VMEM((_C, _D), jnp.float32),
        pltpu.VMEM((_C, _D), jnp.float32),
        pltpu.SemaphoreType.DMA,
        pltpu.SemaphoreType.DMA,
    ),
)
def _sc_gather(x_hbm, idxn_hbm, neg_hbm, idxn_v, buf0, buf1, sem0, sem1):
    wid = lax.axis_index("s") * _NC + lax.axis_index("c")
    nbase = wid * _NPW

    pltpu.sync_copy(idxn_hbm.at[pl.ds(nbase, _NPW)], idxn_v)

    def ngather(c, buf, sem):
        pltpu.async_copy(x_hbm.at[idxn_v.at[pl.ds(c * _C, _C)]], buf, sem)

    def nwait(c, buf, sem):
        pltpu.make_async_copy(x_hbm.at[idxn_v.at[pl.ds(c * _C, _C)]], buf, sem).wait()

    ngather(0, buf0, sem0)

    def nbody(k, carry):
        c0 = 2 * k
        c1 = c0 + 1
        ngather(c1, buf1, sem1)
        nwait(c0, buf0, sem0)
        pltpu.sync_copy(buf0, neg_hbm.at[pl.ds(nbase + c0 * _C, _C)])

        @pl.when(c1 + 1 < _NCH)
        def _():
            ngather(c1 + 1, buf0, sem0)

        nwait(c1, buf1, sem1)
        pltpu.sync_copy(buf1, neg_hbm.at[pl.ds(nbase + c1 * _C, _C)])
        return carry

    lax.fori_loop(0, _NCH // 2, nbody, 0)


def _tc_roll_body(x_ref, tgt_ref, xcopy_ref):
    tgt_ref[pl.ds(0, _BT - 1), :] = x_ref[pl.ds(1, _BT - 1), :]
    tgt_ref[pl.ds(_T - 1, 1), :] = x_ref[pl.ds(0, 1), :]
    tgt_ref[pl.ds(_BT - 1, 1), :] = x_ref[pl.ds(_T, 1), :]
    xcopy_ref[...] = x_ref[...]


_tc_roll = pl.pallas_call(
    _tc_roll_body,
    out_shape=(
        jax.ShapeDtypeStruct((_BT, _D), jnp.float32),
        jax.ShapeDtypeStruct((_BT, _D), jnp.float32),
    ),
)


def kernel(x):
    B, T, D = x.shape
    tszs = jnp.repeat(jnp.arange(T), _NNEG)
    neg = jax.random.randint(jax.random.key(42), (B, _NNEG * T), 0, T - 1)
    neg = jnp.where(neg >= tszs[None, :], neg + 1, neg)
    src_t = jnp.where(neg == T - 1, 0, neg + 1)
    src = src_t + jnp.arange(B)[:, None] * T
    idxn = src.reshape(B, T, _NNEG).transpose(2, 0, 1).reshape(-1)
    idxn = idxn.astype(jnp.int32)

    x_flat = x.reshape(_BT, D)
    negs = _sc_gather(x_flat, idxn)
    tgt, xc = _tc_roll(x_flat)
    return (xc.reshape(B, T, D), tgt.reshape(B, T, D),
            negs.reshape(_NNEG, B, T, D))
